# baseline (speedup 1.0000x reference)
import functools

import jax
import jax.numpy as jnp
from jax import lax
from jax.experimental import pallas as pl
from jax.experimental.pallas import tpu as pltpu
from jax.experimental.pallas import tpu_sc as plsc

N = 10000
E = 320000
F = 128
NG = 64
NCLS = 10
NLAYERS = 3
EPS = 1e-5

NSC = 2
NTILE = 16
NW = NSC * NTILE
EPW = E // NW
CH = 128
NCHUNK = 80
G = 16
NGRP = NCHUNK // G
EPWP = NCHUNK * CH
EPAD = EPWP - EPW
NP = 10240
RPT = NP // NTILE


def _edge_body(x_hbm, src_hbm, dst_hbm, zeros_hbm, out_hbm,
               didx_v, sb0, sb1, r0, r1, agg_sh, g0, g1, i0, i1):
    c = lax.axis_index("c")
    s = lax.axis_index("s")
    w = c * NTILE + s
    sbuf = [sb0, sb1]
    rows = [r0, r1]
    gsem = [g0, g1]
    isem = [i0, i1]

    def start_idx(b, j):
        pltpu.async_copy(src_hbm.at[w, j], sbuf[b], isem[b])

    def wait_idx(b):
        pltpu.make_async_copy(src_hbm.at[w, 0], sbuf[b], isem[b]).wait()

    def start_gather(b):
        pltpu.async_copy(x_hbm.at[sbuf[b]], rows[b], gsem[b])

    def wait_gather(b):
        pltpu.make_async_copy(x_hbm.at[sbuf[b]], rows[b], gsem[b]).wait()

    def scatter(b, k):
        pltpu.sync_copy(rows[b], agg_sh.at[didx_v.at[k]], add=True)

    pltpu.sync_copy(zeros_hbm.at[pl.ds(s * RPT, RPT)],
                    agg_sh.at[pl.ds(s * RPT, RPT)])
    plsc.subcore_barrier()

    start_idx(0, 0)
    start_idx(1, 1)

    def pair(j0, k):
        wait_idx(0)
        start_gather(0)
        wait_gather(0)
        start_idx(0, j0 + 2)
        scatter(0, k)
        wait_idx(1)
        start_gather(1)
        wait_gather(1)
        start_idx(1, j0 + 3)
        scatter(1, k + 1)

    def group_body(g, carry):
        j0 = g * G
        pltpu.sync_copy(dst_hbm.at[w, pl.ds(j0, G)], didx_v)
        for k in range(0, G, 2):
            pair(j0 + k, k)
        return carry

    lax.fori_loop(0, NGRP - 1, group_body, 0)

    j0g = (NGRP - 1) * G
    pltpu.sync_copy(dst_hbm.at[w, pl.ds(j0g, G)], didx_v)
    for k in range(0, G - 2, 2):
        pair(j0g + k, k)
    wait_idx(0)
    start_gather(0)
    wait_gather(0)
    scatter(0, G - 2)
    wait_idx(1)
    start_gather(1)
    wait_gather(1)
    scatter(1, G - 1)

    plsc.subcore_barrier()
    pltpu.sync_copy(agg_sh.at[pl.ds(s * RPT, RPT)],
                    out_hbm.at[c, pl.ds(s * RPT, RPT)])


_edge_call = pl.kernel(
    _edge_body,
    out_type=jax.ShapeDtypeStruct((NSC, NP, F), jnp.float32),
    mesh=plsc.VectorSubcoreMesh(core_axis_name="c", subcore_axis_name="s"),
    scratch_types=[
        pltpu.VMEM((G, CH), jnp.int32),
        pltpu.VMEM((CH,), jnp.int32),
        pltpu.VMEM((CH,), jnp.int32),
        pltpu.VMEM((CH, F), jnp.float32),
        pltpu.VMEM((CH, F), jnp.float32),
        pltpu.VMEM_SHARED((NP, F), jnp.float32),
        pltpu.SemaphoreType.DMA,
        pltpu.SemaphoreType.DMA,
        pltpu.SemaphoreType.DMA,
        pltpu.SemaphoreType.DMA,
    ],
)


def _embed_body(h_ref, we_ref, be_ref, o_ref):
    o_ref[...] = (jnp.dot(h_ref[...], we_ref[...],
                          preferred_element_type=jnp.float32) + be_ref[...])


_embed_call = pl.pallas_call(
    _embed_body,
    out_shape=jax.ShapeDtypeStruct((N, F), jnp.float32),
)


def _layer_body(x_ref, p_ref, w_ref, b_ref, g_ref, bt_ref, o_ref):
    agg = p_ref[0, :N, :] + p_ref[1, :N, :]
    z = 2.0 * x_ref[...] + agg
    y = jnp.dot(z, w_ref[...], preferred_element_type=jnp.float32) + b_ref[...]
    mean = jnp.mean(y, axis=0, keepdims=True)
    d = y - mean
    var = jnp.mean(d * d, axis=0, keepdims=True)
    yn = d * lax.rsqrt(var + EPS) * g_ref[...] + bt_ref[...]
    o_ref[...] = jnp.maximum(yn, 0.0)


_layer_call = pl.pallas_call(
    _layer_body,
    out_shape=jax.ShapeDtypeStruct((N, F), jnp.float32),
)


def _pool_body(x_ref, batch_ref, wfc_ref, bfc_ref, o_ref):
    gids = lax.broadcasted_iota(jnp.int32, (NG, N), 0)
    onehot = (gids == batch_ref[...]).astype(jnp.float32)
    pooled = jnp.dot(onehot, x_ref[...], preferred_element_type=jnp.float32)
    o_ref[...] = (jnp.dot(pooled, wfc_ref[...],
                          preferred_element_type=jnp.float32) + bfc_ref[...])


_pool_call = pl.pallas_call(
    _pool_body,
    out_shape=jax.ShapeDtypeStruct((NG, NCLS), jnp.float32),
)


def kernel(h, edge_index, pair_info, batch, W_emb, b_emb, W, b, gamma, beta,
           Wfc, bfc):
    srcw = pair_info[0].reshape(NW, EPW)
    dstw = pair_info[1].reshape(NW, EPW)
    pad_src = jnp.zeros((NW, EPAD), jnp.int32)
    pad_dst = jnp.broadcast_to(
        N + (jnp.arange(EPAD, dtype=jnp.int32) % (NP - N)), (NW, EPAD))
    src = jnp.concatenate([srcw, pad_src], axis=1).reshape(NW, NCHUNK, CH)
    dst = jnp.concatenate([dstw, pad_dst], axis=1).reshape(NW, NCHUNK, CH)
    zeros = jnp.zeros((NP, F), jnp.float32)
    x = _embed_call(h, W_emb, b_emb.reshape(1, F))
    for l in range(NLAYERS):
        parts = _edge_call(x, src, dst, zeros)
        x = _layer_call(x, parts, W[l], b[l].reshape(1, F),
                        gamma[l].reshape(1, F), beta[l].reshape(1, F))
    return _pool_call(x, batch.reshape(1, N), Wfc, bfc.reshape(1, NCLS))


# speedup vs baseline: 1.1611x; 1.1611x over previous
import functools

import jax
import jax.numpy as jnp
from jax import lax
from jax.experimental import pallas as pl
from jax.experimental.pallas import tpu as pltpu
from jax.experimental.pallas import tpu_sc as plsc

N = 10000
E = 320000
F = 128
NG = 64
NCLS = 10
NLAYERS = 3
EPS = 1e-5

NSC = 2
NTILE = 16
NW = NSC * NTILE
EPW = E // NW
CH = 128
NCHUNK = 80
G = 16
NGRP = NCHUNK // G
EPWP = NCHUNK * CH
EPAD = EPWP - EPW
NP = 10240
RPT = NP // NTILE


def _edge_body(x_hbm, src_hbm, dst_hbm, zeros_hbm, out_hbm,
               didx_v, sb0, sb1, r0, r1, agg_sh, g0, g1, i0, i1):
    c = lax.axis_index("c")
    s = lax.axis_index("s")
    w = c * NTILE + s
    sbuf = [sb0, sb1]
    rows = [r0, r1]
    gsem = [g0, g1]
    isem = [i0, i1]

    def start_idx(b, j):
        pltpu.async_copy(src_hbm.at[w, j], sbuf[b], isem[b])

    def wait_idx(b):
        pltpu.make_async_copy(src_hbm.at[w, 0], sbuf[b], isem[b]).wait()

    def start_gather(b):
        pltpu.async_copy(x_hbm.at[sbuf[b]], rows[b], gsem[b])

    def wait_gather(b):
        pltpu.make_async_copy(x_hbm.at[sbuf[b]], rows[b], gsem[b]).wait()

    def scatter(b, j):
        pltpu.sync_copy(rows[b], agg_sh.at[didx_v.at[j]], add=True)

    pltpu.sync_copy(zeros_hbm.at[pl.ds(s * RPT, RPT)],
                    agg_sh.at[pl.ds(s * RPT, RPT)])
    pltpu.sync_copy(dst_hbm.at[w], didx_v)
    plsc.subcore_barrier()

    start_idx(0, 0)
    wait_idx(0)
    start_gather(0)
    start_idx(1, 1)

    def pair(j0, carry):
        wait_idx(1)
        start_gather(1)
        start_idx(0, j0 + 2)
        wait_gather(0)
        scatter(0, j0)
        wait_idx(0)
        start_gather(0)
        start_idx(1, j0 + 3)
        wait_gather(1)
        scatter(1, j0 + 1)
        return carry

    lax.fori_loop(0, NCHUNK // 2 - 1, lambda i, cc: pair(2 * i, cc), 0)

    wait_idx(1)
    start_gather(1)
    wait_gather(0)
    scatter(0, NCHUNK - 2)
    wait_gather(1)
    scatter(1, NCHUNK - 1)

    plsc.subcore_barrier()
    pltpu.sync_copy(agg_sh.at[pl.ds(s * RPT, RPT)],
                    out_hbm.at[c, pl.ds(s * RPT, RPT)])


_edge_call = pl.kernel(
    _edge_body,
    out_type=jax.ShapeDtypeStruct((NSC, NP, F), jnp.float32),
    mesh=plsc.VectorSubcoreMesh(core_axis_name="c", subcore_axis_name="s"),
    scratch_types=[
        pltpu.VMEM((NCHUNK, CH), jnp.int32),
        pltpu.VMEM((CH,), jnp.int32),
        pltpu.VMEM((CH,), jnp.int32),
        pltpu.VMEM((CH, F), jnp.float32),
        pltpu.VMEM((CH, F), jnp.float32),
        pltpu.VMEM_SHARED((NP, F), jnp.float32),
        pltpu.SemaphoreType.DMA,
        pltpu.SemaphoreType.DMA,
        pltpu.SemaphoreType.DMA,
        pltpu.SemaphoreType.DMA,
    ],
)


def _embed_body(h_ref, we_ref, be_ref, o_ref):
    o_ref[...] = (jnp.dot(h_ref[...], we_ref[...],
                          preferred_element_type=jnp.float32) + be_ref[...])


_embed_call = pl.pallas_call(
    _embed_body,
    out_shape=jax.ShapeDtypeStruct((N, F), jnp.float32),
)


def _layer_body(x_ref, p_ref, w_ref, b_ref, g_ref, bt_ref, o_ref):
    agg = p_ref[0, :N, :] + p_ref[1, :N, :]
    z = 2.0 * x_ref[...] + agg
    y = jnp.dot(z, w_ref[...], preferred_element_type=jnp.float32) + b_ref[...]
    mean = jnp.mean(y, axis=0, keepdims=True)
    d = y - mean
    var = jnp.mean(d * d, axis=0, keepdims=True)
    yn = d * lax.rsqrt(var + EPS) * g_ref[...] + bt_ref[...]
    o_ref[...] = jnp.maximum(yn, 0.0)


_layer_call = pl.pallas_call(
    _layer_body,
    out_shape=jax.ShapeDtypeStruct((N, F), jnp.float32),
)


def _pool_body(x_ref, batch_ref, wfc_ref, bfc_ref, o_ref):
    gids = lax.broadcasted_iota(jnp.int32, (NG, N), 0)
    onehot = (gids == batch_ref[...]).astype(jnp.float32)
    pooled = jnp.dot(onehot, x_ref[...], preferred_element_type=jnp.float32)
    o_ref[...] = (jnp.dot(pooled, wfc_ref[...],
                          preferred_element_type=jnp.float32) + bfc_ref[...])


_pool_call = pl.pallas_call(
    _pool_body,
    out_shape=jax.ShapeDtypeStruct((NG, NCLS), jnp.float32),
)


def kernel(h, edge_index, pair_info, batch, W_emb, b_emb, W, b, gamma, beta,
           Wfc, bfc):
    srcw = pair_info[0].reshape(NW, EPW)
    dstw = pair_info[1].reshape(NW, EPW)
    pad_src = jnp.zeros((NW, EPAD), jnp.int32)
    pad_dst = jnp.broadcast_to(
        N + (jnp.arange(EPAD, dtype=jnp.int32) % (NP - N)), (NW, EPAD))
    src = jnp.concatenate([srcw, pad_src], axis=1).reshape(NW, NCHUNK, CH)
    dst = jnp.concatenate([dstw, pad_dst], axis=1).reshape(NW, NCHUNK, CH)
    zeros = jnp.zeros((NP, F), jnp.float32)
    x = _embed_call(h, W_emb, b_emb.reshape(1, F))
    for l in range(NLAYERS):
        parts = _edge_call(x, src, dst, zeros)
        x = _layer_call(x, parts, W[l], b[l].reshape(1, F),
                        gamma[l].reshape(1, F), beta[l].reshape(1, F))
    return _pool_call(x, batch.reshape(1, N), Wfc, bfc.reshape(1, NCLS))
